# Initial kernel scaffold; baseline (speedup 1.0000x reference)
import jax
import jax.numpy as jnp
from jax.experimental import pallas as pl


def kernel(input_ids, token_embeds):
    raise NotImplementedError("write your pallas kernel here")


# speedup vs baseline: 1.6668x; 1.6668x over previous
import functools

import jax
import jax.numpy as jnp
from jax import lax
from jax.experimental import pallas as pl
from jax.experimental.pallas import tpu as pltpu
from jax.experimental.pallas import tpu_sc as plsc

_EMBED_DIM = 2048
_NUM_CORES = 2
_NUM_SUBCORES = 16
_NUM_WORKERS = _NUM_CORES * _NUM_SUBCORES
_CHUNK = 16


def _gather_call(ids_flat, token_embeds, num_tokens):
    bpw = num_tokens // _NUM_WORKERS
    nch = bpw // _CHUNK
    mesh = plsc.VectorSubcoreMesh(core_axis_name="core", subcore_axis_name="subcore")

    @functools.partial(
        pl.kernel,
        out_type=jax.ShapeDtypeStruct((num_tokens, _EMBED_DIM), token_embeds.dtype),
        mesh=mesh,
        scratch_types=[
            pltpu.VMEM((bpw,), jnp.int32),
            pltpu.VMEM((_CHUNK, _EMBED_DIM), jnp.float32),
            pltpu.VMEM((_CHUNK, _EMBED_DIM), jnp.float32),
            pltpu.SemaphoreType.DMA,
            pltpu.SemaphoreType.DMA,
            pltpu.SemaphoreType.DMA,
            pltpu.SemaphoreType.DMA,
        ],
    )
    def gather_kernel(tab_hbm, idx_hbm, out_hbm, idx_v, buf_a, buf_b, ga, gb, oa, ob):
        wid = lax.axis_index("subcore") * _NUM_CORES + lax.axis_index("core")
        base = wid * bpw
        pltpu.sync_copy(idx_hbm.at[pl.ds(base, bpw)], idx_v)

        def start_gather(c, buf, sem):
            pltpu.make_async_copy(
                tab_hbm.at[idx_v.at[pl.ds(c * _CHUNK, _CHUNK)]], buf, sem
            ).start()

        def wait_gather(c, buf, sem):
            pltpu.make_async_copy(
                tab_hbm.at[idx_v.at[pl.ds(c * _CHUNK, _CHUNK)]], buf, sem
            ).wait()

        def start_out(c, buf, sem):
            pltpu.make_async_copy(
                buf, out_hbm.at[pl.ds(base + c * _CHUNK, _CHUNK)], sem
            ).start()

        def wait_out(c, buf, sem):
            pltpu.make_async_copy(
                buf, out_hbm.at[pl.ds(base + c * _CHUNK, _CHUNK)], sem
            ).wait()

        start_gather(0, buf_a, ga)
        start_gather(1, buf_b, gb)

        @pl.loop(0, nch - 2, step=2)
        def _(c):
            wait_gather(c, buf_a, ga)
            start_out(c, buf_a, oa)
            wait_gather(c + 1, buf_b, gb)
            start_out(c + 1, buf_b, ob)
            wait_out(c, buf_a, oa)
            start_gather(c + 2, buf_a, ga)
            wait_out(c + 1, buf_b, ob)
            start_gather(c + 3, buf_b, gb)

        wait_gather(nch - 2, buf_a, ga)
        start_out(nch - 2, buf_a, oa)
        wait_gather(nch - 1, buf_b, gb)
        start_out(nch - 1, buf_b, ob)
        wait_out(nch - 2, buf_a, oa)
        wait_out(nch - 1, buf_b, ob)

    return gather_kernel(token_embeds, ids_flat)


def kernel(input_ids, token_embeds):
    batch, seq_len = input_ids.shape
    num_tokens = batch * seq_len
    ids_flat = input_ids.astype(jnp.int32).reshape(num_tokens)
    out = _gather_call(ids_flat, token_embeds, num_tokens)
    return out.reshape(batch, seq_len, token_embeds.shape[1])


# speedup vs baseline: 1.7130x; 1.0277x over previous
import functools

import jax
import jax.numpy as jnp
from jax import lax
from jax.experimental import pallas as pl
from jax.experimental.pallas import tpu as pltpu
from jax.experimental.pallas import tpu_sc as plsc

_EMBED_DIM = 2048
_NUM_CORES = 2
_NUM_SUBCORES = 16
_NUM_WORKERS = _NUM_CORES * _NUM_SUBCORES
_CHUNK = 8
_NBUF = 4


def _gather_call(ids_flat, token_embeds, num_tokens):
    bpw = num_tokens // _NUM_WORKERS
    nch = bpw // _CHUNK
    assert nch % _NBUF == 0
    mesh = plsc.VectorSubcoreMesh(core_axis_name="core", subcore_axis_name="subcore")

    @functools.partial(
        pl.kernel,
        out_type=jax.ShapeDtypeStruct((num_tokens, _EMBED_DIM), token_embeds.dtype),
        mesh=mesh,
        scratch_types=(
            [pltpu.VMEM((bpw,), jnp.int32)]
            + [pltpu.VMEM((_CHUNK, _EMBED_DIM), jnp.float32) for _ in range(_NBUF)]
            + [pltpu.SemaphoreType.DMA for _ in range(2 * _NBUF)]
        ),
    )
    def gather_kernel(tab_hbm, idx_hbm, out_hbm, idx_v, *scratch):
        bufs = scratch[:_NBUF]
        gsems = scratch[_NBUF : 2 * _NBUF]
        osems = scratch[2 * _NBUF :]
        wid = lax.axis_index("subcore") * _NUM_CORES + lax.axis_index("core")
        base = wid * bpw
        pltpu.sync_copy(idx_hbm.at[pl.ds(base, bpw)], idx_v)

        def gather_cp(c, b):
            return pltpu.make_async_copy(
                tab_hbm.at[idx_v.at[pl.ds(c * _CHUNK, _CHUNK)]], bufs[b], gsems[b]
            )

        def out_cp(c, b):
            return pltpu.make_async_copy(
                bufs[b], out_hbm.at[pl.ds(base + c * _CHUNK, _CHUNK)], osems[b]
            )

        for b in range(_NBUF):
            gather_cp(b, b).start()

        @pl.loop(0, nch - _NBUF, step=_NBUF)
        def _(c):
            for b in range(_NBUF):
                gather_cp(c + b, b).wait()
                out_cp(c + b, b).start()
            for b in range(_NBUF):
                out_cp(c + b, b).wait()
                gather_cp(c + _NBUF + b, b).start()

        for b in range(_NBUF):
            gather_cp(nch - _NBUF + b, b).wait()
            out_cp(nch - _NBUF + b, b).start()
        for b in range(_NBUF):
            out_cp(nch - _NBUF + b, b).wait()

    return gather_kernel(token_embeds, ids_flat)


def kernel(input_ids, token_embeds):
    batch, seq_len = input_ids.shape
    num_tokens = batch * seq_len
    ids_flat = input_ids.astype(jnp.int32).reshape(num_tokens)
    out = _gather_call(ids_flat, token_embeds, num_tokens)
    return out.reshape(batch, seq_len, token_embeds.shape[1])
